# Initial kernel scaffold; baseline (speedup 1.0000x reference)
import jax
import jax.numpy as jnp
from jax.experimental import pallas as pl


def kernel(x, pre_bias, W_enc, latent_bias, W_dec):
    raise NotImplementedError("write your pallas kernel here")


# speedup vs baseline: 10.7388x; 10.7388x over previous
import jax
import jax.numpy as jnp
from jax import lax
from jax.experimental import pallas as pl

TOPK = 64


def _enc_body(x_ref, w_ref, pb_ref, lb_ref, out_ref):
    xc = x_ref[...] - pb_ref[...]
    out_ref[...] = lax.dot_general(
        xc, w_ref[...], (((1,), (1,)), ((), ())),
        preferred_element_type=jnp.float32) + lb_ref[...]


def _thr_body(lat_ref, out_ref):
    acts = jnp.maximum(lat_ref[...], 0.0)
    keys = lax.bitcast_convert_type(acts, jnp.int32)

    def body(i, t):
        b = (30 - i).astype(jnp.int32)
        cand = t | lax.shift_left(jnp.int32(1), b)
        cnt = jnp.sum((keys >= cand).astype(jnp.int32), axis=1, keepdims=True)
        return jnp.where(cnt >= TOPK, cand, t)

    t0 = jnp.zeros((acts.shape[0], 1), jnp.int32)
    t = lax.fori_loop(0, 31, body, t0)
    out_ref[...] = jnp.broadcast_to(t, out_ref.shape)


def _dec_body(lat_ref, thr_ref, w_ref, pb_ref, out_ref):
    kidx = pl.program_id(1)
    acts = jnp.maximum(lat_ref[...], 0.0)
    keys = lax.bitcast_convert_type(acts, jnp.int32)
    thr = thr_ref[...][:, 0:1]
    masked = jnp.where(keys >= thr, acts, 0.0)
    part = lax.dot_general(
        masked, w_ref[...], (((1,), (1,)), ((), ())),
        preferred_element_type=jnp.float32)

    @pl.when(kidx == 0)
    def _():
        out_ref[...] = part + pb_ref[...]

    @pl.when(kidx != 0)
    def _():
        out_ref[...] += part


def kernel(x, pre_bias, W_enc, latent_bias, W_dec):
    M, D = x.shape
    N = W_enc.shape[0]
    pb2 = pre_bias.reshape(1, D)
    lb2 = latent_bias.reshape(1, N)

    BM1 = min(512, M)
    BN1 = min(2048, N)
    latents = pl.pallas_call(
        _enc_body,
        grid=(N // BN1, M // BM1),
        in_specs=[
            pl.BlockSpec((BM1, D), lambda n, m: (m, 0)),
            pl.BlockSpec((BN1, D), lambda n, m: (n, 0)),
            pl.BlockSpec((1, D), lambda n, m: (0, 0)),
            pl.BlockSpec((1, BN1), lambda n, m: (0, n)),
        ],
        out_specs=pl.BlockSpec((BM1, BN1), lambda n, m: (m, n)),
        out_shape=jax.ShapeDtypeStruct((M, N), jnp.float32),
    )(x, W_enc, pb2, lb2)

    TM = min(256, M)
    thr = pl.pallas_call(
        _thr_body,
        grid=(M // TM,),
        in_specs=[pl.BlockSpec((TM, N), lambda m: (m, 0))],
        out_specs=pl.BlockSpec((TM, 128), lambda m: (m, 0)),
        out_shape=jax.ShapeDtypeStruct((M, 128), jnp.int32),
    )(latents)

    BM2 = min(1024, M)
    BK2 = min(1024, N)
    recons = pl.pallas_call(
        _dec_body,
        grid=(M // BM2, N // BK2),
        in_specs=[
            pl.BlockSpec((BM2, BK2), lambda m, k: (m, k)),
            pl.BlockSpec((BM2, 128), lambda m, k: (m, 0)),
            pl.BlockSpec((D, BK2), lambda m, k: (0, k)),
            pl.BlockSpec((1, D), lambda m, k: (0, 0)),
        ],
        out_specs=pl.BlockSpec((BM2, D), lambda m, k: (m, 0)),
        out_shape=jax.ShapeDtypeStruct((M, D), jnp.float32),
    )(latents, thr, W_dec, pb2)

    return (recons, latents)


# speedup vs baseline: 14.4238x; 1.3432x over previous
import jax
import jax.numpy as jnp
from jax import lax
from jax.experimental import pallas as pl

TOPK = 64


def _enc_body(x_ref, w_ref, pb_ref, lb_ref, out_ref):
    xc = x_ref[...] - pb_ref[...]
    out_ref[...] = lax.dot_general(
        xc, w_ref[...], (((1,), (1,)), ((), ())),
        preferred_element_type=jnp.float32) + lb_ref[...]


def _thr_body(lat_ref, out_ref):
    keys = lax.bitcast_convert_type(lat_ref[...], jnp.int32)
    rows = keys.shape[0]
    rowmax = jnp.max(keys, axis=1, keepdims=True)

    def body(i, state):
        t, done = state
        b = (30 - i).astype(jnp.int32)
        cand = t | lax.shift_left(jnp.int32(1), b)
        feasible = (rowmax >= cand).astype(jnp.int32)
        active = (1 - done) * feasible

        def do_pass():
            return jnp.sum((keys >= cand).astype(jnp.int32), axis=1,
                           keepdims=True)

        cnt = lax.cond(jnp.max(active) > 0, do_pass,
                       lambda: jnp.zeros((rows, 1), jnp.int32))
        cnt = cnt * feasible
        take = (1 - done) * (cnt >= TOPK).astype(jnp.int32)
        t = jnp.where(take > 0, cand, t)
        done = jnp.where((take > 0) & (cnt == TOPK), 1, done)
        return (t, done)

    t0 = jnp.zeros((rows, 1), jnp.int32)
    d0 = jnp.zeros((rows, 1), jnp.int32)
    t, _ = lax.fori_loop(0, 31, body, (t0, d0))
    out_ref[...] = jnp.broadcast_to(t, out_ref.shape)


def _dec_body(lat_ref, thr_ref, w_ref, pb_ref, out_ref):
    kidx = pl.program_id(1)
    lat = lat_ref[...]
    keys = lax.bitcast_convert_type(lat, jnp.int32)
    thr = thr_ref[...][:, 0:1]
    masked = jnp.where(keys >= thr, lat, 0.0)
    part = lax.dot_general(
        masked, w_ref[...], (((1,), (1,)), ((), ())),
        preferred_element_type=jnp.float32)

    @pl.when(kidx == 0)
    def _():
        out_ref[...] = part + pb_ref[...]

    @pl.when(kidx != 0)
    def _():
        out_ref[...] += part


def kernel(x, pre_bias, W_enc, latent_bias, W_dec):
    M, D = x.shape
    N = W_enc.shape[0]
    pb2 = pre_bias.reshape(1, D)
    lb2 = latent_bias.reshape(1, N)

    BM1 = min(512, M)
    BN1 = min(2048, N)
    latents = pl.pallas_call(
        _enc_body,
        grid=(N // BN1, M // BM1),
        in_specs=[
            pl.BlockSpec((BM1, D), lambda n, m: (m, 0)),
            pl.BlockSpec((BN1, D), lambda n, m: (n, 0)),
            pl.BlockSpec((1, D), lambda n, m: (0, 0)),
            pl.BlockSpec((1, BN1), lambda n, m: (0, n)),
        ],
        out_specs=pl.BlockSpec((BM1, BN1), lambda n, m: (m, n)),
        out_shape=jax.ShapeDtypeStruct((M, N), jnp.float32),
    )(x, W_enc, pb2, lb2)

    TM = min(256, M)
    thr = pl.pallas_call(
        _thr_body,
        grid=(M // TM,),
        in_specs=[pl.BlockSpec((TM, N), lambda m: (m, 0))],
        out_specs=pl.BlockSpec((TM, 128), lambda m: (m, 0)),
        out_shape=jax.ShapeDtypeStruct((M, 128), jnp.int32),
    )(latents)

    BM2 = min(1024, M)
    BK2 = min(1024, N)
    recons = pl.pallas_call(
        _dec_body,
        grid=(M // BM2, N // BK2),
        in_specs=[
            pl.BlockSpec((BM2, BK2), lambda m, k: (m, k)),
            pl.BlockSpec((BM2, 128), lambda m, k: (m, 0)),
            pl.BlockSpec((D, BK2), lambda m, k: (0, k)),
            pl.BlockSpec((1, D), lambda m, k: (0, 0)),
        ],
        out_specs=pl.BlockSpec((BM2, D), lambda m, k: (m, 0)),
        out_shape=jax.ShapeDtypeStruct((M, D), jnp.float32),
    )(latents, thr, W_dec, pb2)

    return (recons, latents)
